# Initial kernel scaffold; baseline (speedup 1.0000x reference)
import jax
import jax.numpy as jnp
from jax.experimental import pallas as pl


def kernel(x, in_degree, out_degree, atom_table, in_table, out_table, graph_token):
    raise NotImplementedError("write your pallas kernel here")


# speedup vs baseline: 4.9013x; 4.9013x over previous
import functools

import jax
import jax.numpy as jnp
from jax import lax
from jax.experimental import pallas as pl
from jax.experimental.pallas import tpu as pltpu
from jax.experimental.pallas import tpu_sc as plsc

NUM_ATOMS = 4608
NUM_IN_DEG = 512
NUM_OUT_DEG = 512
H = 768
B = 256
N = 128
F = 9
IPN = F + 2
NC = 2
NS = 16
NW = NC * NS
GPW = B // NW
C = 4
KPG = N // C
IPC = C * IPN
LANES = H // 16


def _sum_chunk(buf, ost):
    def col(v, carry):
        base = v * 16
        for i in range(C):
            acc = buf[i * IPN, pl.ds(base, 16)]
            for j in range(1, IPN):
                acc = acc + buf[i * IPN + j, pl.ds(base, 16)]
            ost[i, pl.ds(base, 16)] = acc
        return carry
    lax.fori_loop(0, LANES, col, 0, unroll=False)


def _graph_node_feature_kernel(table_hbm, idx_hbm, tok_hbm, out_hbm,
                               idx_v, buf0, buf1, ost0, ost1, tok_v,
                               sg0, sg1, ss0, ss1):
    wid = lax.axis_index("s") * NC + lax.axis_index("c")

    pltpu.sync_copy(tok_hbm, tok_v)

    def graph_body(g, carry):
        gid = wid * GPW + g
        pltpu.sync_copy(idx_hbm.at[gid], idx_v)
        pltpu.sync_copy(tok_v, out_hbm.at[gid * (N + 1)])

        pltpu.async_copy(table_hbm.at[idx_v.at[0]], buf0, sg0)

        def pair(t, c2):
            k0 = 2 * t
            row0 = gid * (N + 1) + 1 + C * k0

            pltpu.async_copy(table_hbm.at[idx_v.at[k0 + 1]], buf1, sg1)

            pltpu.make_async_copy(table_hbm.at[idx_v.at[0]], buf0, sg0).wait()

            @pl.when(t > 0)
            def _wait_prev_store0():
                pltpu.make_async_copy(ost0, out_hbm.at[pl.ds(0, C)], ss0).wait()

            _sum_chunk(buf0, ost0)
            pltpu.async_copy(ost0, out_hbm.at[pl.ds(row0, C)], ss0)

            @pl.when(t < KPG // 2 - 1)
            def _fire_next():
                pltpu.async_copy(table_hbm.at[idx_v.at[k0 + 2]], buf0, sg0)

            pltpu.make_async_copy(table_hbm.at[idx_v.at[0]], buf1, sg1).wait()

            @pl.when(t > 0)
            def _wait_prev_store1():
                pltpu.make_async_copy(ost1, out_hbm.at[pl.ds(0, C)], ss1).wait()

            _sum_chunk(buf1, ost1)
            pltpu.async_copy(ost1, out_hbm.at[pl.ds(row0 + C, C)], ss1)
            return c2

        lax.fori_loop(0, KPG // 2, pair, 0, unroll=False)

        pltpu.make_async_copy(ost0, out_hbm.at[pl.ds(0, C)], ss0).wait()
        pltpu.make_async_copy(ost1, out_hbm.at[pl.ds(0, C)], ss1).wait()
        return carry

    lax.fori_loop(0, GPW, graph_body, 0, unroll=False)


@jax.jit
def _run(table, idx, tok):
    mesh = plsc.VectorSubcoreMesh(core_axis_name="c", subcore_axis_name="s")
    return pl.kernel(
        _graph_node_feature_kernel,
        out_type=jax.ShapeDtypeStruct((B * (N + 1), H), jnp.float32),
        mesh=mesh,
        scratch_types=[
            pltpu.VMEM((KPG, IPC), jnp.int32),
            pltpu.VMEM((IPC, H), jnp.float32),
            pltpu.VMEM((IPC, H), jnp.float32),
            pltpu.VMEM((C, H), jnp.float32),
            pltpu.VMEM((C, H), jnp.float32),
            pltpu.VMEM((H,), jnp.float32),
            pltpu.SemaphoreType.DMA,
            pltpu.SemaphoreType.DMA,
            pltpu.SemaphoreType.DMA,
            pltpu.SemaphoreType.DMA,
        ],
        compiler_params=pltpu.CompilerParams(use_tc_tiling_on_sc=False),
    )(table, idx, tok)


def kernel(x, in_degree, out_degree, atom_table, in_table, out_table, graph_token):
    x = x.astype(jnp.int32)
    in_degree = in_degree.astype(jnp.int32)
    out_degree = out_degree.astype(jnp.int32)
    idx = jnp.concatenate(
        [
            x,
            (in_degree + (NUM_ATOMS + 1))[..., None],
            (out_degree + (NUM_ATOMS + 1 + NUM_IN_DEG))[..., None],
        ],
        axis=-1,
    ).reshape(B, KPG, IPC)
    table = jnp.concatenate([atom_table, in_table, out_table], axis=0)
    out = _run(table, idx, graph_token.reshape(H))
    return out.reshape(B, N + 1, H)
